# baseline (speedup 1.0000x reference)
import jax
import jax.numpy as jnp
from jax import lax
from jax.experimental import pallas as pl
from jax.experimental.pallas import tpu as pltpu
from jax.experimental.pallas import tpu_sc as plsc

NUM_CORES = 2
NUM_SUBCORES = 16
NUM_WORKERS = NUM_CORES * NUM_SUBCORES
LANES = 16
BATCH = 16384
DIM = 64
WIDE = 2 * DIM
B_PER_W = BATCH // NUM_WORKERS
CHUNK = 128
NCHUNK = B_PER_W // CHUNK
GROUPS = CHUNK // LANES
NBUF = 3
TR_STRIDE = LANES + 1


def _gmf_body(uidx_hbm, iidx_hbm, utab_hbm, itab_hbm, w_hbm, b_hbm,
              out_hbm,
              idxr_u, idxr_i, idxg_u, idxg_i, rows_u, rows_i,
              w_v, b_v, tr_v, out_v,
              sem0, sem1, sem2):
    sems = [sem0, sem1, sem2]
    wid = lax.axis_index("s") * NUM_CORES + lax.axis_index("c")
    base = wid * B_PER_W

    pltpu.sync_copy(w_hbm, w_v)
    pltpu.sync_copy(b_hbm, b_v)

    for k in range(NCHUNK):
        pltpu.sync_copy(uidx_hbm.at[pl.ds(base + k * CHUNK, CHUNK)],
                        idxr_u.at[k])
        pltpu.sync_copy(iidx_hbm.at[pl.ds(base + k * CHUNK, CHUNK)],
                        idxr_i.at[k])
    for k in range(NCHUNK):
        for j in range(CHUNK // LANES):
            sl = pl.ds(j * LANES, LANES)
            idxg_u[k, sl] = lax.shift_right_logical(idxr_u[k, sl], 1)
            idxg_i[k, sl] = lax.shift_right_logical(idxr_i[k, sl], 1)

    def fire(k):
        s = sems[k % NBUF]
        hu = pltpu.async_copy(utab_hbm.at[idxg_u.at[k]], rows_u.at[k % NBUF], s)
        hi = pltpu.async_copy(itab_hbm.at[idxg_i.at[k]], rows_i.at[k % NBUF], s)
        return hu, hi

    handles = {k: fire(k) for k in range(min(NBUF, NCHUNK))}

    wv = [w_v[pl.ds(c * LANES, LANES)] for c in range(4)]
    bias = b_v[...]
    lane = lax.iota(jnp.int32, LANES)
    lane_c = [lane + c * LANES for c in range(4)]
    lane_tr = lane * TR_STRIDE

    for k in range(NCHUNK):
        hu, hi = handles[k]
        hu.wait()
        hi.wait()
        uk = rows_u.at[k % NBUF]
        ik = rows_i.at[k % NBUF]

        def group_body(g, carry, uk=uk, ik=ik, k=k):
            kvec = jnp.full((LANES,), k, jnp.int32)
            for bi in range(LANES):
                b = g * LANES + bi
                row = jnp.full((LANES,), 0, jnp.int32) + b
                hu_b = lax.shift_left(
                    plsc.load_gather(idxr_u, [kvec, row]) & 1, 6)
                hi_b = lax.shift_left(
                    plsc.load_gather(idxr_i, [kvec, row]) & 1, 6)
                s = None
                for c in range(4):
                    uvec = plsc.load_gather(uk, [row, hu_b + lane_c[c]])
                    ivec = plsc.load_gather(ik, [row, hi_b + lane_c[c]])
                    t = uvec * ivec * wv[c]
                    s = t if s is None else s + t
                tr_v[pl.ds(bi * TR_STRIDE, LANES)] = s
            acc = bias
            for d in range(LANES):
                acc = acc + plsc.load_gather(tr_v, [lane_tr + d])
            out16 = 1.0 / (1.0 + jnp.exp(-acc))
            out_v[pl.ds(k * CHUNK + g * LANES, LANES)] = out16
            return carry

        lax.fori_loop(0, GROUPS, group_body, 0)
        if k + NBUF < NCHUNK:
            handles[k + NBUF] = fire(k + NBUF)

    pltpu.sync_copy(out_v, out_hbm.at[pl.ds(base, B_PER_W)])


def _build():
    mesh = plsc.VectorSubcoreMesh(core_axis_name="c", subcore_axis_name="s")
    return pl.kernel(
        _gmf_body,
        mesh=mesh,
        compiler_params=pltpu.CompilerParams(needs_layout_passes=False),
        out_type=jax.ShapeDtypeStruct((BATCH,), jnp.float32),
        scratch_types=[
            pltpu.VMEM((NCHUNK, CHUNK), jnp.int32),
            pltpu.VMEM((NCHUNK, CHUNK), jnp.int32),
            pltpu.VMEM((NCHUNK, CHUNK), jnp.int32),
            pltpu.VMEM((NCHUNK, CHUNK), jnp.int32),
            pltpu.VMEM((NBUF, CHUNK, WIDE), jnp.float32),
            pltpu.VMEM((NBUF, CHUNK, WIDE), jnp.float32),
            pltpu.VMEM((DIM,), jnp.float32),
            pltpu.VMEM((LANES,), jnp.float32),
            pltpu.VMEM((LANES * TR_STRIDE,), jnp.float32),
            pltpu.VMEM((B_PER_W,), jnp.float32),
            pltpu.SemaphoreType.DMA,
            pltpu.SemaphoreType.DMA,
            pltpu.SemaphoreType.DMA,
        ],
    )


def kernel(user_indices, item_indices, user_table, item_table, W, b):
    uidx = user_indices.astype(jnp.int32)
    iidx = item_indices.astype(jnp.int32)
    ut2 = user_table.reshape(-1, WIDE)
    it2 = item_table.reshape(-1, WIDE)
    w_flat = W.reshape(DIM).astype(jnp.float32)
    b16 = jnp.broadcast_to(b.astype(jnp.float32), (LANES,))
    out = _build()(uidx, iidx, ut2, it2, w_flat, b16)
    return out.reshape(BATCH, 1)


# speedup vs baseline: 1.0105x; 1.0105x over previous
import jax
import jax.numpy as jnp
from jax import lax
from jax.experimental import pallas as pl
from jax.experimental.pallas import tpu as pltpu
from jax.experimental.pallas import tpu_sc as plsc

NUM_CORES = 2
NUM_SUBCORES = 16
NUM_WORKERS = NUM_CORES * NUM_SUBCORES
LANES = 16
BATCH = 16384
DIM = 64
B_PER_W = BATCH // NUM_WORKERS
CHUNK = 128
NCHUNK = B_PER_W // CHUNK
GROUPS = CHUNK // LANES
NBUF = 3
TR_STRIDE = LANES + 1


def _gmf_body(uidx_hbm, iidx_hbm, utab_hbm, itab_hbm, w_hbm, b_hbm,
              out_hbm,
              idx_u, idx_i, rows_u, rows_i,
              w_v, b_v, tr_v, out_v,
              sem0, sem1, sem2):
    sems = [sem0, sem1, sem2]
    wid = lax.axis_index("s") * NUM_CORES + lax.axis_index("c")
    base = wid * B_PER_W

    pltpu.sync_copy(w_hbm, w_v)
    pltpu.sync_copy(b_hbm, b_v)

    for k in range(NCHUNK):
        pltpu.sync_copy(uidx_hbm.at[pl.ds(base + k * CHUNK, CHUNK)],
                        idx_u.at[k])
        pltpu.sync_copy(iidx_hbm.at[pl.ds(base + k * CHUNK, CHUNK)],
                        idx_i.at[k])

    def fire(k):
        s = sems[k % NBUF]
        hu = pltpu.async_copy(utab_hbm.at[idx_u.at[k]], rows_u.at[k % NBUF], s)
        hi = pltpu.async_copy(itab_hbm.at[idx_i.at[k]], rows_i.at[k % NBUF], s)
        return hu, hi

    handles = {k: fire(k) for k in range(min(NBUF, NCHUNK))}

    wv = [w_v[pl.ds(c * LANES, LANES)] for c in range(4)]
    bias = b_v[...]
    lane = lax.iota(jnp.int32, LANES)
    lane_c = [lane + c * LANES for c in range(4)]
    lane_tr = lane * TR_STRIDE

    for k in range(NCHUNK):
        hu, hi = handles[k]
        hu.wait()
        hi.wait()
        uk = rows_u.at[k % NBUF]
        ik = rows_i.at[k % NBUF]

        def group_body(g, carry, uk=uk, ik=ik, k=k):
            for bi in range(LANES):
                row = g * LANES + bi + jnp.full((LANES,), 0, jnp.int32)
                s = None
                for c in range(4):
                    uvec = plsc.load_gather(uk, [row, lane_c[c]])
                    ivec = plsc.load_gather(ik, [row, lane_c[c]])
                    t = uvec * ivec * wv[c]
                    s = t if s is None else s + t
                tr_v[pl.ds(bi * TR_STRIDE, LANES)] = s
            acc = bias
            for d in range(LANES):
                acc = acc + plsc.load_gather(tr_v, [lane_tr + d])
            out16 = 1.0 / (1.0 + jnp.exp(-acc))
            out_v[pl.ds(k * CHUNK + g * LANES, LANES)] = out16
            return carry

        lax.fori_loop(0, GROUPS, group_body, 0)
        if k + NBUF < NCHUNK:
            handles[k + NBUF] = fire(k + NBUF)

    pltpu.sync_copy(out_v, out_hbm.at[pl.ds(base, B_PER_W)])


def _build():
    mesh = plsc.VectorSubcoreMesh(core_axis_name="c", subcore_axis_name="s")
    return pl.kernel(
        _gmf_body,
        mesh=mesh,
        compiler_params=pltpu.CompilerParams(needs_layout_passes=False,
                                             use_tc_tiling_on_sc=False),
        out_type=jax.ShapeDtypeStruct((BATCH,), jnp.float32),
        scratch_types=[
            pltpu.VMEM((NCHUNK, CHUNK), jnp.int32),
            pltpu.VMEM((NCHUNK, CHUNK), jnp.int32),
            pltpu.VMEM((NBUF, CHUNK, DIM), jnp.float32),
            pltpu.VMEM((NBUF, CHUNK, DIM), jnp.float32),
            pltpu.VMEM((DIM,), jnp.float32),
            pltpu.VMEM((LANES,), jnp.float32),
            pltpu.VMEM((LANES * TR_STRIDE,), jnp.float32),
            pltpu.VMEM((B_PER_W,), jnp.float32),
            pltpu.SemaphoreType.DMA,
            pltpu.SemaphoreType.DMA,
            pltpu.SemaphoreType.DMA,
        ],
    )


def kernel(user_indices, item_indices, user_table, item_table, W, b):
    uidx = user_indices.astype(jnp.int32)
    iidx = item_indices.astype(jnp.int32)
    w_flat = W.reshape(DIM).astype(jnp.float32)
    b16 = jnp.broadcast_to(b.astype(jnp.float32), (LANES,))
    out = _build()(uidx, iidx, user_table, item_table, w_flat, b16)
    return out.reshape(BATCH, 1)


# speedup vs baseline: 1.5796x; 1.5632x over previous
import jax
import jax.numpy as jnp
from jax import lax
from jax.experimental import pallas as pl
from jax.experimental.pallas import tpu as pltpu
from jax.experimental.pallas import tpu_sc as plsc

NUM_CORES = 2
NUM_SUBCORES = 16
NUM_WORKERS = NUM_CORES * NUM_SUBCORES
LANES = 16
BATCH = 16384
DIM = 64
B_PER_W = BATCH // NUM_WORKERS
CHUNK = 128
NCHUNK = B_PER_W // CHUNK
GROUPS = CHUNK // LANES
NBUF = 3
TR_STRIDE = LANES + 1


def _gmf_body(uidx_hbm, iidx_hbm, utab_hbm, itab_hbm, w_hbm, b_hbm,
              out_hbm,
              idx_u, idx_i, rows_u, rows_i,
              w_v, b_v, tr_v, out_v,
              sem0, sem1, sem2):
    sems = [sem0, sem1, sem2]
    wid = lax.axis_index("s") * NUM_CORES + lax.axis_index("c")
    base = wid * B_PER_W

    pltpu.sync_copy(w_hbm, w_v)
    pltpu.sync_copy(b_hbm, b_v)

    for k in range(NCHUNK):
        pltpu.sync_copy(uidx_hbm.at[pl.ds(base + k * CHUNK, CHUNK)],
                        idx_u.at[k])
        pltpu.sync_copy(iidx_hbm.at[pl.ds(base + k * CHUNK, CHUNK)],
                        idx_i.at[k])

    def fire(k):
        s = sems[k % NBUF]
        ub = rows_u.at[k % NBUF]
        ib = rows_i.at[k % NBUF]

        def fire_group(g, carry):
            vu = idx_u[k, pl.ds(g * LANES, LANES)]
            vi = idx_i[k, pl.ds(g * LANES, LANES)]
            for j in range(LANES):
                row = g * LANES + j
                pltpu.async_copy(utab_hbm.at[pl.ds(vu[j], 1), :],
                                 ub.at[pl.ds(row, 1), :], s)
                pltpu.async_copy(itab_hbm.at[pl.ds(vi[j], 1), :],
                                 ib.at[pl.ds(row, 1), :], s)
            return carry

        lax.fori_loop(0, GROUPS, fire_group, 0)

    def drain(k):
        s = sems[k % NBUF]

        def wait_one(j, carry):
            pltpu.make_async_copy(utab_hbm.at[pl.ds(0, 1), :],
                                  rows_u.at[k % NBUF].at[pl.ds(0, 1), :],
                                  s).wait()
            return carry

        lax.fori_loop(0, 2 * CHUNK, wait_one, 0)

    for k in range(min(NBUF, NCHUNK)):
        fire(k)

    wv = [w_v[pl.ds(c * LANES, LANES)] for c in range(4)]
    bias = b_v[...]
    lane = lax.iota(jnp.int32, LANES)
    lane_c = [lane + c * LANES for c in range(4)]
    lane_tr = lane * TR_STRIDE

    for k in range(NCHUNK):
        drain(k)
        uk = rows_u.at[k % NBUF]
        ik = rows_i.at[k % NBUF]

        def group_body(g, carry, uk=uk, ik=ik, k=k):
            for bi in range(LANES):
                row = g * LANES + bi + jnp.full((LANES,), 0, jnp.int32)
                s = None
                for c in range(4):
                    uvec = plsc.load_gather(uk, [row, lane_c[c]])
                    ivec = plsc.load_gather(ik, [row, lane_c[c]])
                    t = uvec * ivec * wv[c]
                    s = t if s is None else s + t
                tr_v[pl.ds(bi * TR_STRIDE, LANES)] = s
            acc = bias
            for d in range(LANES):
                acc = acc + plsc.load_gather(tr_v, [lane_tr + d])
            out16 = 1.0 / (1.0 + jnp.exp(-acc))
            out_v[pl.ds(k * CHUNK + g * LANES, LANES)] = out16
            return carry

        lax.fori_loop(0, GROUPS, group_body, 0)
        if k + NBUF < NCHUNK:
            fire(k + NBUF)

    pltpu.sync_copy(out_v, out_hbm.at[pl.ds(base, B_PER_W)])


def _build():
    mesh = plsc.VectorSubcoreMesh(core_axis_name="c", subcore_axis_name="s")
    return pl.kernel(
        _gmf_body,
        mesh=mesh,
        compiler_params=pltpu.CompilerParams(needs_layout_passes=False),
        out_type=jax.ShapeDtypeStruct((BATCH,), jnp.float32),
        scratch_types=[
            pltpu.VMEM((NCHUNK, CHUNK), jnp.int32),
            pltpu.VMEM((NCHUNK, CHUNK), jnp.int32),
            pltpu.VMEM((NBUF, CHUNK, DIM), jnp.float32),
            pltpu.VMEM((NBUF, CHUNK, DIM), jnp.float32),
            pltpu.VMEM((DIM,), jnp.float32),
            pltpu.VMEM((LANES,), jnp.float32),
            pltpu.VMEM((LANES * TR_STRIDE,), jnp.float32),
            pltpu.VMEM((B_PER_W,), jnp.float32),
            pltpu.SemaphoreType.DMA,
            pltpu.SemaphoreType.DMA,
            pltpu.SemaphoreType.DMA,
        ],
    )


def kernel(user_indices, item_indices, user_table, item_table, W, b):
    uidx = user_indices.astype(jnp.int32)
    iidx = item_indices.astype(jnp.int32)
    w_flat = W.reshape(DIM).astype(jnp.float32)
    b16 = jnp.broadcast_to(b.astype(jnp.float32), (LANES,))
    out = _build()(uidx, iidx, user_table, item_table, w_flat, b16)
    return out.reshape(BATCH, 1)


# speedup vs baseline: 1.5859x; 1.0040x over previous
import jax
import jax.numpy as jnp
from jax import lax
from jax.experimental import pallas as pl
from jax.experimental.pallas import tpu as pltpu
from jax.experimental.pallas import tpu_sc as plsc

NUM_CORES = 2
NUM_SUBCORES = 16
NUM_WORKERS = NUM_CORES * NUM_SUBCORES
LANES = 16
BATCH = 16384
DIM = 64
B_PER_W = BATCH // NUM_WORKERS
CHUNK = 128
NCHUNK = B_PER_W // CHUNK
GROUPS = CHUNK // LANES
NBUF = 3
TR_STRIDE = LANES + 1


def _gmf_body(uidx_hbm, iidx_hbm, utab_hbm, itab_hbm, w_hbm, b_hbm,
              out_hbm,
              idx_u, idx_i, rows_u, rows_i,
              w_v, b_v, tr_v, out_v,
              sem0, sem1, sem2, sem3, sem4, sem5):
    sems_u = [sem0, sem1, sem2]
    sems_i = [sem3, sem4, sem5]
    wid = lax.axis_index("s") * NUM_CORES + lax.axis_index("c")
    base = wid * B_PER_W

    pltpu.sync_copy(w_hbm, w_v)
    pltpu.sync_copy(b_hbm, b_v)

    for k in range(NCHUNK):
        pltpu.sync_copy(uidx_hbm.at[pl.ds(base + k * CHUNK, CHUNK)],
                        idx_u.at[k])
        pltpu.sync_copy(iidx_hbm.at[pl.ds(base + k * CHUNK, CHUNK)],
                        idx_i.at[k])

    def fire(k):
        su = sems_u[k % NBUF]
        si = sems_i[k % NBUF]
        ub = rows_u.at[k % NBUF]
        ib = rows_i.at[k % NBUF]

        def fire_group(g, carry):
            vu = idx_u[k, pl.ds(g * LANES, LANES)]
            vi = idx_i[k, pl.ds(g * LANES, LANES)]
            for j in range(LANES):
                row = g * LANES + j
                pltpu.async_copy(utab_hbm.at[pl.ds(vu[j], 1), :],
                                 ub.at[pl.ds(row, 1), :], su)
                pltpu.async_copy(itab_hbm.at[pl.ds(vi[j], 1), :],
                                 ib.at[pl.ds(row, 1), :], si)
            return carry

        lax.fori_loop(0, GROUPS, fire_group, 0)

    def drain(k):
        def wait_one(j, carry):
            pltpu.make_async_copy(utab_hbm.at[pl.ds(0, 1), :],
                                  rows_u.at[k % NBUF].at[pl.ds(0, 1), :],
                                  sems_u[k % NBUF]).wait()
            pltpu.make_async_copy(itab_hbm.at[pl.ds(0, 1), :],
                                  rows_i.at[k % NBUF].at[pl.ds(0, 1), :],
                                  sems_i[k % NBUF]).wait()
            return carry

        lax.fori_loop(0, CHUNK, wait_one, 0)

    for k in range(min(NBUF, NCHUNK)):
        fire(k)

    wv = [w_v[pl.ds(c * LANES, LANES)] for c in range(4)]
    bias = b_v[...]
    lane = lax.iota(jnp.int32, LANES)
    lane_c = [lane + c * LANES for c in range(4)]
    lane_tr = lane * TR_STRIDE

    for k in range(NCHUNK):
        drain(k)
        uk = rows_u.at[k % NBUF]
        ik = rows_i.at[k % NBUF]

        def group_body(g, carry, uk=uk, ik=ik, k=k):
            for bi in range(LANES):
                row = g * LANES + bi + jnp.full((LANES,), 0, jnp.int32)
                s = None
                for c in range(4):
                    uvec = plsc.load_gather(uk, [row, lane_c[c]])
                    ivec = plsc.load_gather(ik, [row, lane_c[c]])
                    t = uvec * ivec * wv[c]
                    s = t if s is None else s + t
                tr_v[pl.ds(bi * TR_STRIDE, LANES)] = s
            acc = bias
            for d in range(LANES):
                acc = acc + plsc.load_gather(tr_v, [lane_tr + d])
            out16 = 1.0 / (1.0 + jnp.exp(-acc))
            out_v[pl.ds(k * CHUNK + g * LANES, LANES)] = out16
            return carry

        lax.fori_loop(0, GROUPS, group_body, 0)
        if k + NBUF < NCHUNK:
            fire(k + NBUF)

    pltpu.sync_copy(out_v, out_hbm.at[pl.ds(base, B_PER_W)])


def _build():
    mesh = plsc.VectorSubcoreMesh(core_axis_name="c", subcore_axis_name="s")
    return pl.kernel(
        _gmf_body,
        mesh=mesh,
        compiler_params=pltpu.CompilerParams(needs_layout_passes=False),
        out_type=jax.ShapeDtypeStruct((BATCH,), jnp.float32),
        scratch_types=[
            pltpu.VMEM((NCHUNK, CHUNK), jnp.int32),
            pltpu.VMEM((NCHUNK, CHUNK), jnp.int32),
            pltpu.VMEM((NBUF, CHUNK, DIM), jnp.float32),
            pltpu.VMEM((NBUF, CHUNK, DIM), jnp.float32),
            pltpu.VMEM((DIM,), jnp.float32),
            pltpu.VMEM((LANES,), jnp.float32),
            pltpu.VMEM((LANES * TR_STRIDE,), jnp.float32),
            pltpu.VMEM((B_PER_W,), jnp.float32),
            pltpu.SemaphoreType.DMA,
            pltpu.SemaphoreType.DMA,
            pltpu.SemaphoreType.DMA,
            pltpu.SemaphoreType.DMA,
            pltpu.SemaphoreType.DMA,
            pltpu.SemaphoreType.DMA,
        ],
    )


def kernel(user_indices, item_indices, user_table, item_table, W, b):
    uidx = user_indices.astype(jnp.int32)
    iidx = item_indices.astype(jnp.int32)
    w_flat = W.reshape(DIM).astype(jnp.float32)
    b16 = jnp.broadcast_to(b.astype(jnp.float32), (LANES,))
    out = _build()(uidx, iidx, user_table, item_table, w_flat, b16)
    return out.reshape(BATCH, 1)
